# baseline (speedup 1.0000x reference)
import math

import jax
import jax.numpy as jnp
import numpy as np
from jax.experimental import pallas as pl

DIM = 1024
FEAT = 4
SCALE = math.sqrt(DIM)
LO = 64


def _make_tables():
    d = np.arange(DIM)
    freq = np.exp(-(2 * (d // 2)).astype(np.float64) * (math.log(10000.0) / DIM))
    even = (d % 2) == 0

    n_hi = 4096 // LO
    hi_angle = (LO * np.arange(n_hi, dtype=np.float64))[:, None] * freq[None, :]
    p_hi = np.where(even[None, :], np.sin(hi_angle), np.cos(hi_angle))
    q_hi = np.where(even[None, :], np.cos(hi_angle), -np.sin(hi_angle))

    lo_angle = np.arange(LO, dtype=np.float64)[:, None] * freq[None, :]
    c_lo = np.repeat(np.cos(lo_angle), FEAT, axis=0)
    s_lo = np.repeat(np.sin(lo_angle), FEAT, axis=0)

    return (
        jnp.asarray(p_hi.astype(np.float32))[:, None, :],
        jnp.asarray(q_hi.astype(np.float32))[:, None, :],
        jnp.asarray(c_lo.astype(np.float32)),
        jnp.asarray(s_lo.astype(np.float32)),
    )


_P_HI, _Q_HI, _C_LO, _S_LO = _make_tables()


def _block_kernel(emb_ref, p_ref, q_ref, cl_ref, sl_ref, out_ref):
    pe = p_ref[0] * cl_ref[...] + q_ref[0] * sl_ref[...]
    out_ref[...] = emb_ref[...] * SCALE + pe


def kernel(emb):
    seq, feat, dim = emb.shape
    rows = seq * feat
    block_rows = LO * feat
    emb2 = emb.reshape(rows, dim)
    grid = (seq // LO,)
    out = pl.pallas_call(
        _block_kernel,
        grid=grid,
        in_specs=[
            pl.BlockSpec((block_rows, dim), lambda i: (i, 0)),
            pl.BlockSpec((1, 1, dim), lambda i: (i, 0, 0)),
            pl.BlockSpec((1, 1, dim), lambda i: (i, 0, 0)),
            pl.BlockSpec((block_rows, dim), lambda i: (0, 0)),
            pl.BlockSpec((block_rows, dim), lambda i: (0, 0)),
        ],
        out_specs=pl.BlockSpec((block_rows, dim), lambda i: (i, 0)),
        out_shape=jax.ShapeDtypeStruct((rows, dim), emb.dtype),
    )(emb2, _P_HI, _Q_HI, _C_LO, _S_LO)
    return out.reshape(seq, feat, dim)


# speedup vs baseline: 4.5995x; 4.5995x over previous
import math

import jax
import jax.numpy as jnp
import numpy as np
from jax.experimental import pallas as pl

DIM = 1024
SCALE = math.sqrt(DIM)
LO = 256


def _make_tables(seq):
    d = np.arange(DIM)
    freq = np.exp(-(2 * (d // 2)).astype(np.float64) * (math.log(10000.0) / DIM))
    even = (d % 2) == 0

    n_hi = seq // LO
    hi_angle = (LO * np.arange(n_hi, dtype=np.float64))[:, None] * freq[None, :]
    p_hi = np.where(even[None, :], np.sin(hi_angle), np.cos(hi_angle))
    q_hi = np.where(even[None, :], np.cos(hi_angle), -np.sin(hi_angle))

    lo_angle = np.arange(LO, dtype=np.float64)[:, None] * freq[None, :]
    c_lo = np.cos(lo_angle)
    s_lo = np.sin(lo_angle)

    return (
        p_hi.astype(np.float32)[:, None, :],
        q_hi.astype(np.float32)[:, None, :],
        c_lo.astype(np.float32),
        s_lo.astype(np.float32),
    )


_TABLES = _make_tables(4096)


def _block_kernel(emb_ref, p_ref, q_ref, cl_ref, sl_ref, out_ref):
    pe = p_ref[0] * cl_ref[...] + q_ref[0] * sl_ref[...]
    out_ref[...] = emb_ref[...] * SCALE + pe[:, None, :]


def kernel(emb):
    seq, feat, dim = emb.shape
    grid = (seq // LO,)
    return pl.pallas_call(
        _block_kernel,
        grid=grid,
        in_specs=[
            pl.BlockSpec((LO, feat, dim), lambda i: (i, 0, 0)),
            pl.BlockSpec((1, 1, dim), lambda i: (i, 0, 0)),
            pl.BlockSpec((1, 1, dim), lambda i: (i, 0, 0)),
            pl.BlockSpec((LO, dim), lambda i: (0, 0)),
            pl.BlockSpec((LO, dim), lambda i: (0, 0)),
        ],
        out_specs=pl.BlockSpec((LO, feat, dim), lambda i: (i, 0, 0)),
        out_shape=jax.ShapeDtypeStruct((seq, feat, dim), emb.dtype),
    )(emb, *_TABLES)
